# Initial kernel scaffold; baseline (speedup 1.0000x reference)
import jax
import jax.numpy as jnp
from jax.experimental import pallas as pl


def kernel(edge_index, edge_type, subj, rel, init_embed, init_rel, W1_neigh, W1_loop, W1_rel, b1, W2_neigh, W2_loop, W2_rel, b2):
    raise NotImplementedError("write your pallas kernel here")


# speedup vs baseline: 2.5937x; 2.5937x over previous
import jax
import jax.numpy as jnp
from jax import lax
from jax.experimental import pallas as pl
from jax.experimental.pallas import tpu as pltpu
from jax.experimental.pallas import tpu_sc as plsc

N_ENT = 10000
N_REL = 200
DIM = 128
N_EDGE = 320000
BATCH = 1024

NC = 2
NS = 16
HALF = N_ENT // NC
ACC_ROWS = HALF + 8

EC = 128
EPT = 20096
E_PAD = EPT * NS
CPT = EPT // EC
ZR = 1000


def _edge_pass_body(src_hbm, dst_hbm, et_hbm, x_hbm, r_hbm, z_hbm, z1_hbm,
                    ones_hbm,
                    out_hbm, deg_hbm,
                    srcv, etv, dstv, xbuf, rbuf, onesv, acc, dacc,
                    sem1, sem2):
    cid = lax.axis_index("c")
    sid = lax.axis_index("s")

    @pl.when(sid < 5)
    def _zero():
        sl = pl.ds(sid * ZR, ZR)
        pltpu.sync_copy(z_hbm, acc.at[sl])

    @pl.when(sid == 5)
    def _zero_deg():
        pltpu.sync_copy(z1_hbm, dacc)

    pltpu.sync_copy(ones_hbm, onesv)

    plsc.subcore_barrier()

    base0 = sid * EPT

    def _chunk(j, carry):
        base = base0 + j * EC
        pltpu.sync_copy(src_hbm.at[pl.ds(base, EC)], srcv)
        pltpu.sync_copy(et_hbm.at[pl.ds(base, EC)], etv)
        pltpu.sync_copy(dst_hbm.at[cid, pl.ds(base, EC)], dstv)
        cp1 = pltpu.async_copy(x_hbm.at[srcv], xbuf, sem1)
        cp2 = pltpu.async_copy(r_hbm.at[etv], rbuf, sem2)
        cp1.wait()
        cp2.wait()

        def _mul(e, c2):
            for dd in range(8):
                sl = pl.ds(dd * 16, 16)
                xbuf[e, sl] = xbuf[e, sl] * rbuf[e, sl]
            return c2
        lax.fori_loop(0, EC, _mul, 0)

        pltpu.sync_copy(xbuf, acc.at[dstv], add=True)
        pltpu.sync_copy(onesv, dacc.at[dstv], add=True)
        return carry

    lax.fori_loop(0, CPT, _chunk, 0)

    plsc.subcore_barrier()

    @pl.when(sid < 5)
    def _copyout():
        sl = pl.ds(sid * ZR, ZR)
        pltpu.sync_copy(acc.at[sl], out_hbm.at[cid, sl])

    @pl.when(sid == 5)
    def _copyout_deg():
        pltpu.sync_copy(dacc, deg_hbm.at[cid])


def _make_edge_pass():
    mesh = plsc.VectorSubcoreMesh(core_axis_name="c", subcore_axis_name="s")
    out_type = (jax.ShapeDtypeStruct((NC, HALF, DIM), jnp.float32),
                jax.ShapeDtypeStruct((NC, ACC_ROWS), jnp.float32))
    scratch = [
        pltpu.VMEM((EC,), jnp.int32),
        pltpu.VMEM((EC,), jnp.int32),
        pltpu.VMEM((EC,), jnp.int32),
        pltpu.VMEM((EC, DIM), jnp.float32),
        pltpu.VMEM((EC, DIM), jnp.float32),
        pltpu.VMEM((EC,), jnp.float32),
        pltpu.VMEM_SHARED((ACC_ROWS, DIM), jnp.float32),
        pltpu.VMEM_SHARED((ACC_ROWS,), jnp.float32),
        pltpu.SemaphoreType.DMA,
        pltpu.SemaphoreType.DMA,
    ]
    return pl.kernel(
        _edge_pass_body,
        out_type=out_type,
        mesh=mesh,
        scratch_types=scratch,
    )


def _dense_body(parts_ref, deg_ref, x_ref, wn_ref, wl_ref, b_ref, o_ref):
    s = parts_ref[0]
    deg = jnp.maximum(deg_ref[0], 1.0)
    t = jnp.dot(s / deg, wn_ref[...], preferred_element_type=jnp.float32)
    t += jnp.dot(x_ref[...], wl_ref[...], preferred_element_type=jnp.float32)
    o_ref[...] = jnp.tanh(t + b_ref[...])


def _dense(parts, degp, x, wn, wl, b):
    blk = 1000
    nb = HALF // blk
    grid = (N_ENT // blk,)
    return pl.pallas_call(
        _dense_body,
        grid=grid,
        in_specs=[
            pl.BlockSpec((1, blk, DIM), lambda i: (i // nb, i % nb, 0)),
            pl.BlockSpec((1, blk, 1), lambda i: (i // nb, i % nb, 0)),
            pl.BlockSpec((blk, DIM), lambda i: (i, 0)),
            pl.BlockSpec((DIM, DIM), lambda i: (0, 0)),
            pl.BlockSpec((DIM, DIM), lambda i: (0, 0)),
            pl.BlockSpec((1, DIM), lambda i: (0, 0)),
        ],
        out_specs=pl.BlockSpec((blk, DIM), lambda i: (i, 0)),
        out_shape=jax.ShapeDtypeStruct((N_ENT, DIM), jnp.float32),
    )(parts, degp, x, wn, wl, b)


def _rel_body(r_ref, w1_ref, w2_ref, r1_ref, r2_ref):
    r1 = jnp.dot(r_ref[...], w1_ref[...], preferred_element_type=jnp.float32)
    r1_ref[...] = r1
    r2_ref[...] = jnp.dot(r1, w2_ref[...], preferred_element_type=jnp.float32)


def _rel_chain(r0, w1, w2):
    return pl.pallas_call(
        _rel_body,
        out_shape=(jax.ShapeDtypeStruct((N_REL, DIM), jnp.float32),
                   jax.ShapeDtypeStruct((N_REL, DIM), jnp.float32)),
    )(r0, w1, w2)


def _gather_body(x_hbm, r_hbm, subj_hbm, rel_hbm, sub_out, rel_out,
                 sidx, ridx, srows, rrows, sem):
    cid = lax.axis_index("c")
    sid = lax.axis_index("s")
    wid = sid * NC + cid
    bpw = BATCH // (NC * NS)
    base = wid * bpw
    pltpu.sync_copy(subj_hbm.at[pl.ds(base, bpw)], sidx)
    pltpu.sync_copy(rel_hbm.at[pl.ds(base, bpw)], ridx)
    cp1 = pltpu.async_copy(x_hbm.at[sidx], srows, sem)
    cp1.wait()
    pltpu.sync_copy(srows, sub_out.at[pl.ds(base, bpw)])
    cp2 = pltpu.async_copy(r_hbm.at[ridx], rrows, sem)
    cp2.wait()
    pltpu.sync_copy(rrows, rel_out.at[pl.ds(base, bpw)])


def _make_gather():
    bpw = BATCH // (NC * NS)
    mesh = plsc.VectorSubcoreMesh(core_axis_name="c", subcore_axis_name="s")
    return pl.kernel(
        _gather_body,
        out_type=(jax.ShapeDtypeStruct((BATCH, DIM), jnp.float32),
                  jax.ShapeDtypeStruct((BATCH, DIM), jnp.float32)),
        mesh=mesh,
        scratch_types=[
            pltpu.VMEM((bpw,), jnp.int32),
            pltpu.VMEM((bpw,), jnp.int32),
            pltpu.VMEM((bpw, DIM), jnp.float32),
            pltpu.VMEM((bpw, DIM), jnp.float32),
            pltpu.SemaphoreType.DMA,
        ],
    )


_edge_pass = _make_edge_pass()
_final_gather = _make_gather()


def kernel(edge_index, edge_type, subj, rel, init_embed, init_rel,
           W1_neigh, W1_loop, W1_rel, b1, W2_neigh, W2_loop, W2_rel, b2):
    src = edge_index[0].astype(jnp.int32)
    dst = edge_index[1].astype(jnp.int32)
    et = edge_type.astype(jnp.int32)
    npad = E_PAD - N_EDGE
    src = jnp.concatenate([src, jnp.zeros((npad,), jnp.int32)])
    dst = jnp.concatenate([dst, jnp.full((npad,), N_ENT, jnp.int32)])
    et = jnp.concatenate([et, jnp.zeros((npad,), jnp.int32)])
    halves = jnp.arange(NC, dtype=jnp.int32)[:, None] * HALF
    d_loc = dst[None, :] - halves
    dst_loc = jnp.where((d_loc >= 0) & (d_loc < HALF), d_loc, HALF)
    zeros = jnp.zeros((ZR, DIM), jnp.float32)
    zeros1 = jnp.zeros((ACC_ROWS,), jnp.float32)
    ones1 = jnp.ones((EC,), jnp.float32)

    parts1, degp = _edge_pass(src, dst_loc, et, init_embed, init_rel,
                              zeros, zeros1, ones1)
    degp = degp[:, :HALF, None]
    r1, r2 = _rel_chain(init_rel, W1_rel, W2_rel)
    x1 = _dense(parts1, degp, init_embed, W1_neigh, W1_loop,
                b1.reshape(1, DIM))
    parts2, _ = _edge_pass(src, dst_loc, et, x1, r1, zeros, zeros1, ones1)
    x2 = _dense(parts2, degp, x1, W2_neigh, W2_loop, b2.reshape(1, DIM))
    sub_emb, rel_emb = _final_gather(x2, r2, subj.astype(jnp.int32),
                                     rel.astype(jnp.int32))
    return (sub_emb, rel_emb, x2)
